# Initial kernel scaffold; baseline (speedup 1.0000x reference)
import jax
import jax.numpy as jnp
from jax.experimental import pallas as pl


def kernel(means, colors, det, opacities, means2d, depths, radius, conics, covars2d):
    raise NotImplementedError("write your pallas kernel here")


# speedup vs baseline: 2.6983x; 2.6983x over previous
import functools

import jax
import jax.numpy as jnp
from jax.experimental import pallas as pl
from jax.experimental.pallas import tpu as pltpu

_K = 512
_NEAR = 0.0
_FAR = 2.0
_WIDTH = 200.0
_HEIGHT = 600.0


def _body(means_r, colors_r, det_r, opac_r, m2d_r, dep_r, rad_r, con_r, cov_r,
          o_means, o_colors, o_m2d, o_dep, o_rad, o_cov, o_con, o_opac,
          o_pack, o_cnt, ms_ref, ds_ref, stag_ref, *, nchunk, nco):
    K = _K
    det = det_r[0, :, 0, :]
    dep = dep_r[0, :, 0, :]
    valid = (det > 0.0) & (dep > _NEAR) & (dep < _FAR)
    mx = m2d_r[0, :, 0, :]
    my = m2d_r[0, :, 1, :]
    rx = rad_r[0, :, 0, :]
    ry = rad_r[0, :, 1, :]
    zero = jnp.float32(0.0)
    rxe = jnp.where(valid, rx, zero)
    rye = jnp.where(valid, ry, zero)
    inside = ((mx + rxe > 0.0) & (mx - rxe < _WIDTH)
              & (my + rye > 0.0) & (my - rye < _HEIGHT))
    maskf = (inside & valid).astype(jnp.float32)
    ms_ref[...] = maskf.reshape(nchunk, 1, K)

    ia = jax.lax.broadcasted_iota(jnp.int32, (K, K), 0)
    ib = jax.lax.broadcasted_iota(jnp.int32, (K, K), 1)
    ut = (ia < ib).astype(jnp.float32)
    ds_ref[...] = jax.lax.dot_general(
        maskf, ut, (((1,), (0,)), ((), ())),
        preferred_element_type=jnp.float32).reshape(nchunk, 1, K)

    pa = jax.lax.broadcasted_iota(jnp.int32, (K, K // 8), 0)
    pb = jax.lax.broadcasted_iota(jnp.int32, (K, K // 8), 1)
    pw = jnp.where((pa // 8) == pb,
                   jnp.left_shift(jnp.int32(1), pa % 8), 0
                   ).astype(jnp.float32)
    o_pack[0] = jax.lax.dot_general(
        maskf, pw, (((1,), (0,)), ((), ())),
        preferred_element_type=jnp.float32).astype(jnp.int32)

    stag_ref[...] = jnp.zeros((nco, 18, K), jnp.float32)

    jr2 = jax.lax.broadcasted_iota(jnp.int32, (2 * K, K), 0)

    def step(i, off):
        mrow = ms_ref[i]
        drow = ds_ref[i].astype(jnp.int32)
        ccnt = jnp.sum(mrow).astype(jnp.int32)
        c_out = off // K
        rem = off % K
        drel = drow + rem
        qt = jnp.where((jnp.broadcast_to(drel, (2 * K, K)) == jr2)
                       & (jnp.broadcast_to(mrow, (2 * K, K)) > 0.5),
                       1.0, 0.0).astype(jnp.float32)
        chunk = jnp.concatenate([
            means_r[0, i], colors_r[0, i], m2d_r[0, i], dep_r[0, i],
            rad_r[0, i], cov_r[0, i], con_r[0, i], opac_r[0, i]],
            axis=0)
        outc = jax.lax.dot_general(
            chunk, qt, (((1,), (1,)), ((), ())),
            preferred_element_type=jnp.float32)
        stag_ref[c_out] = stag_ref[c_out] + outc[:, :K]
        stag_ref[c_out + 1] = stag_ref[c_out + 1] + outc[:, K:]
        return off + ccnt

    total = jax.lax.fori_loop(0, nchunk, step, jnp.int32(0))

    lane = jax.lax.broadcasted_iota(jnp.int32, (18, K), 1)

    def copy_out(j, _):
        merged = jnp.where(j * K + lane >= total, 1.0, stag_ref[j])
        dst = pl.ds(j * K, K)
        o_means[0, :, dst] = merged[0:3]
        o_colors[0, :, dst] = merged[3:6]
        o_m2d[0, :, dst] = merged[6:8]
        o_dep[0, :, dst] = merged[8:9]
        o_rad[0, :, dst] = merged[9:11]
        o_cov[0, :, dst] = merged[11:14]
        o_con[0, :, dst] = merged[14:17]
        o_opac[0, :, dst] = merged[17:18]
        return 0

    jax.lax.fori_loop(0, nco, copy_out, 0)
    o_cnt[...] = jnp.full((1, 1, 1), total, jnp.int32)


@jax.jit
def kernel(means, colors, det, opacities, means2d, depths, radius, conics,
           covars2d):
    B_, C_, N_ = det.shape
    K = _K
    nchunk = -(-N_ // K)
    n_pad = nchunk * K
    nco = nchunk + 1
    n_out = nco * K
    pad = n_pad - N_

    def prep(x):
        x = jnp.pad(x, [(0, 0)] * (x.ndim - 1) + [(0, pad)])
        if x.ndim == 2:
            return x.reshape(1, 1, nchunk, K).transpose(0, 2, 1, 3)
        if x.ndim == 3 and x.shape[1] == C_:
            return x.reshape(C_, 1, nchunk, K).transpose(0, 2, 1, 3)
        if x.ndim == 3:
            return x.reshape(1, x.shape[1], nchunk, K).transpose(0, 2, 1, 3)
        return x.reshape(C_, x.shape[2], nchunk, K).transpose(0, 2, 1, 3)

    means_p = prep(means)
    colors_p = prep(colors)
    det_p = prep(det)
    opac_p = prep(opacities)
    m2d_p = prep(means2d)
    dep_p = prep(depths)
    rad_p = prep(radius)
    con_p = prep(conics)
    cov_p = prep(covars2d)

    def cspec(d):
        return pl.BlockSpec((1, nchunk, d, K), lambda c: (c, 0, 0, 0))

    def bspec(d):
        return pl.BlockSpec((1, nchunk, d, K), lambda c: (0, 0, 0, 0))

    def ospec(d):
        return pl.BlockSpec((1, d, n_out), lambda c: (c, 0, 0))

    out_shapes = (
        jax.ShapeDtypeStruct((C_, 3, n_out), jnp.float32),
        jax.ShapeDtypeStruct((C_, 3, n_out), jnp.float32),
        jax.ShapeDtypeStruct((C_, 2, n_out), jnp.float32),
        jax.ShapeDtypeStruct((C_, 1, n_out), jnp.float32),
        jax.ShapeDtypeStruct((C_, 2, n_out), jnp.float32),
        jax.ShapeDtypeStruct((C_, 3, n_out), jnp.float32),
        jax.ShapeDtypeStruct((C_, 3, n_out), jnp.float32),
        jax.ShapeDtypeStruct((C_, 1, n_out), jnp.float32),
        jax.ShapeDtypeStruct((C_, nchunk, K // 8), jnp.int32),
        jax.ShapeDtypeStruct((C_, 1, 1), jnp.int32),
    )
    out_specs = (
        ospec(3), ospec(3), ospec(2), ospec(1), ospec(2), ospec(3),
        ospec(3), ospec(1),
        pl.BlockSpec((1, nchunk, K // 8), lambda c: (c, 0, 0)),
        pl.BlockSpec((1, 1, 1), lambda c: (c, 0, 0)),
    )

    outs = pl.pallas_call(
        functools.partial(_body, nchunk=nchunk, nco=nco),
        grid=(C_,),
        in_specs=[bspec(3), bspec(3), cspec(1), bspec(1), cspec(2),
                  cspec(1), cspec(2), cspec(3), cspec(3)],
        out_specs=out_specs,
        out_shape=out_shapes,
        scratch_shapes=[pltpu.VMEM((nchunk, 1, K), jnp.float32),
                        pltpu.VMEM((nchunk, 1, K), jnp.float32),
                        pltpu.VMEM((nco, 18, K), jnp.float32)],
    )(means_p, colors_p, det_p, opac_p, m2d_p, dep_p, rad_p, con_p, cov_p)

    (means_o, colors_o, m2d_o, dep_o, rad_o, cov_o, con_o, opac_o,
     pack_o, cnt_o) = outs

    def trim(x):
        return x[None, :, :, :N_]

    filter_uint8 = pack_o.reshape(C_, nchunk * (K // 8))[:, :(N_ + 7) // 8]
    filter_uint8 = filter_uint8.astype(jnp.uint8)[None]
    cnt = cnt_o.reshape(1, C_).astype(jnp.int32)
    return (trim(means_o), trim(colors_o), trim(m2d_o),
            dep_o[None, :, 0, :N_], trim(rad_o), trim(cov_o), trim(con_o),
            opac_o[None, :, 0, :N_], filter_uint8, cnt)
